# baseline (speedup 1.0000x reference)
import functools

import jax
import jax.numpy as jnp
from jax import lax
from jax.experimental import pallas as pl
from jax.experimental.pallas import tpu as pltpu
from jax.experimental.pallas import tpu_sc as plsc

T = 2048
D = 2048
F_MOE = 1024
E = 8
N_SHARED = 2
F_SHARED = F_MOE * N_SHARED
SCALE = 2.5

BLK = 128
PBLK = (2 * T) // BLK + E
P = PBLK * BLK

NC = 2
NS = 16
NW = NC * NS
TPW = T // NW
CH = 32
D2 = D // 2
_HI = 0xFFFF0000
_RB = 0x8000


def _silu(x):
    return x * jax.nn.sigmoid(x)


def _router_body(x_ref, gw_ref, eb_ref, dest_ref, w_ref, blk_ref, nblk_ref,
                 xb_ref, excl_ref, m_ref):
    x = x_ref[...]
    xb = x.astype(jnp.bfloat16)
    xlo = lax.bitcast_convert_type(x[:, :D2], jnp.uint32)
    xhi = lax.bitcast_convert_type(x[:, D2:], jnp.uint32)
    xb_ref[...] = (((xlo + jnp.uint32(_RB)) >> 16)
                   | ((xhi + jnp.uint32(_RB)) & jnp.uint32(_HI)))
    gwb = gw_ref[...].astype(jnp.bfloat16)
    logits = lax.dot_general(
        xb, gwb, (((1,), (1,)), ((), ())),
        preferred_element_type=jnp.float32)
    scores = jax.nn.sigmoid(logits)
    corrected = scores + eb_ref[...]

    iota_e = lax.broadcasted_iota(jnp.int32, (T, E), 1)
    m1 = jnp.max(corrected, axis=1, keepdims=True)
    msk1 = corrected == m1
    sel1 = jnp.min(jnp.where(msk1, iota_e, jnp.int32(E)), axis=1,
                   keepdims=True)
    oh1 = (iota_e == sel1).astype(jnp.float32)
    corrected2 = jnp.where(oh1 > 0, -jnp.inf, corrected)
    m2 = jnp.max(corrected2, axis=1, keepdims=True)
    msk2 = corrected2 == m2
    sel2 = jnp.min(jnp.where(msk2, iota_e, jnp.int32(E)), axis=1,
                   keepdims=True)
    oh2 = (iota_e == sel2).astype(jnp.float32)

    w1 = jnp.sum(oh1 * scores, axis=1, keepdims=True)
    w2 = jnp.sum(oh2 * scores, axis=1, keepdims=True)
    denom = w1 + w2 + jnp.float32(1e-20)
    w1 = w1 / denom * jnp.float32(SCALE)
    w2 = w2 / denom * jnp.float32(SCALE)
    w_ref[...] = jnp.concatenate([w1, w2], axis=1)

    m_ref[...] = oh1 + oh2
    chr_ = 512
    tri = (lax.broadcasted_iota(jnp.int32, (chr_, chr_), 0)
           > lax.broadcasted_iota(jnp.int32, (chr_, chr_), 1)
           ).astype(jnp.float32)

    def body(c, prefix):
        blk = m_ref[pl.ds(c * chr_, chr_), :]
        part = lax.dot_general(
            tri, blk, (((1,), (0,)), ((), ())),
            preferred_element_type=jnp.float32,
            precision=lax.Precision.HIGHEST)
        excl_ref[pl.ds(c * chr_, chr_), :] = part + prefix
        return prefix + jnp.sum(blk, axis=0, keepdims=True)

    counts = lax.fori_loop(0, T // chr_, body, jnp.zeros((1, E), jnp.float32))
    excl = excl_ref[...]

    pc = jnp.ceil(counts / BLK) * BLK
    upper = (lax.broadcasted_iota(jnp.int32, (E, E), 0)
             < lax.broadcasted_iota(jnp.int32, (E, E), 1)).astype(jnp.float32)
    off = lax.dot_general(pc, upper, (((1,), (0,)), ((), ())),
                          preferred_element_type=jnp.float32)

    d1 = jnp.sum(oh1 * (off + excl), axis=1, keepdims=True)
    d2 = jnp.sum(oh2 * (off + excl), axis=1, keepdims=True)
    dest_ref[...] = jnp.concatenate([d1, d2], axis=1).astype(jnp.int32)

    starts = (lax.broadcasted_iota(jnp.int32, (1, PBLK), 1)
              * BLK).astype(jnp.float32)
    offc = jnp.reshape(off, (E, 1))
    be = jnp.sum((starts >= offc).astype(jnp.int32), axis=0,
                 keepdims=True) - 1
    blk_ref[...] = be
    used = jnp.sum(pc) / BLK
    nblk_ref[...] = used.astype(jnp.int32).reshape(1, 1)


def _router(x, gate_w, e_bias):
    return pl.pallas_call(
        _router_body,
        out_shape=(
            jax.ShapeDtypeStruct((T, 2), jnp.int32),
            jax.ShapeDtypeStruct((T, 2), jnp.float32),
            jax.ShapeDtypeStruct((1, PBLK), jnp.int32),
            jax.ShapeDtypeStruct((1, 1), jnp.int32),
            jax.ShapeDtypeStruct((T, D2), jnp.uint32),
        ),
        scratch_shapes=[pltpu.VMEM((T, E), jnp.float32),
                        pltpu.VMEM((T, E), jnp.float32)],
    )(x, gate_w, e_bias.reshape(1, E))


def _sc_mesh():
    return plsc.VectorSubcoreMesh(core_axis_name="c", subcore_axis_name="s",
                                  num_cores=NC, num_subcores=NS)


def _dispatch(x, dest_cm):
    @functools.partial(
        pl.kernel,
        mesh=_sc_mesh(),
        out_type=jax.ShapeDtypeStruct((P, D2), jnp.uint32),
        scratch_types=[
            pltpu.VMEM((CH,), jnp.int32),
            pltpu.VMEM((CH,), jnp.int32),
            pltpu.VMEM((CH, D2), jnp.uint32),
            pltpu.SemaphoreType.DMA,
        ],
    )
    def k(x_hbm, dest_hbm, xs_hbm, idx0_v, idx1_v, rows_v, sem):
        wid = lax.axis_index("s") * NC + lax.axis_index("c")
        base = wid * TPW
        for c in range(TPW // CH):
            off = base + c * CH
            pltpu.sync_copy(dest_hbm.at[0, pl.ds(off, CH)], idx0_v)
            pltpu.sync_copy(dest_hbm.at[1, pl.ds(off, CH)], idx1_v)
            pltpu.sync_copy(x_hbm.at[pl.ds(off, CH)], rows_v)
            a = pltpu.async_copy(rows_v, xs_hbm.at[idx0_v], sem)
            b = pltpu.async_copy(rows_v, xs_hbm.at[idx1_v], sem)
            a.wait()
            b.wait()

    return k(x, dest_cm)


def _gather(outs, dest_cm):
    @functools.partial(
        pl.kernel,
        mesh=_sc_mesh(),
        out_type=(
            jax.ShapeDtypeStruct((T, D2), jnp.uint32),
            jax.ShapeDtypeStruct((T, D2), jnp.uint32),
        ),
        scratch_types=[
            pltpu.VMEM((CH,), jnp.int32),
            pltpu.VMEM((CH, D2), jnp.uint32),
            pltpu.SemaphoreType.DMA,
        ],
    )
    def k(outs_hbm, dest_hbm, g0_hbm, g1_hbm, idx_v, rows_v, sem):
        wid = lax.axis_index("s") * NC + lax.axis_index("c")
        base = wid * TPW
        for c in range(TPW // CH):
            off = base + c * CH
            pltpu.sync_copy(dest_hbm.at[0, pl.ds(off, CH)], idx_v)
            pltpu.async_copy(outs_hbm.at[idx_v], rows_v, sem).wait()
            pltpu.sync_copy(rows_v, g0_hbm.at[pl.ds(off, CH)])
            pltpu.sync_copy(dest_hbm.at[1, pl.ds(off, CH)], idx_v)
            pltpu.async_copy(outs_hbm.at[idx_v], rows_v, sem).wait()
            pltpu.sync_copy(rows_v, g1_hbm.at[pl.ds(off, CH)])

    return k(outs, dest_cm)


def _gemm_body(be_ref, nb_ref, x_ref, wgu_ref, wdn_ref, out_ref):
    i = pl.program_id(0)

    @pl.when(i < nb_ref[0])
    def _():
        v = x_ref[...]
        xlo = lax.bitcast_convert_type(v << 16,
                                       jnp.float32).astype(jnp.bfloat16)
        xhi = lax.bitcast_convert_type(v & jnp.uint32(_HI),
                                       jnp.float32).astype(jnp.bfloat16)
        wgu = wgu_ref[0]
        gu = jnp.dot(xlo, wgu[:D2].astype(jnp.bfloat16),
                     preferred_element_type=jnp.float32)
        gu += jnp.dot(xhi, wgu[D2:].astype(jnp.bfloat16),
                      preferred_element_type=jnp.float32)
        g = gu[:, :F_MOE]
        u = gu[:, F_MOE:]
        act = (_silu(g) * u).astype(jnp.bfloat16)
        o = jnp.dot(act, wdn_ref[0].astype(jnp.bfloat16),
                    preferred_element_type=jnp.float32)
        olo = lax.bitcast_convert_type(o[:, :D2], jnp.uint32)
        ohi = lax.bitcast_convert_type(o[:, D2:], jnp.uint32)
        out_ref[...] = (((olo + jnp.uint32(_RB)) >> 16)
                        | ((ohi + jnp.uint32(_RB)) & jnp.uint32(_HI)))


def _grouped_gemm(be, nb, xs, w_gate_up, w_down):
    grid_spec = pltpu.PrefetchScalarGridSpec(
        num_scalar_prefetch=2,
        grid=(PBLK,),
        in_specs=[
            pl.BlockSpec((BLK, D2),
                         lambda i, be, nb: (jnp.minimum(i, nb[0] - 1), 0)),
            pl.BlockSpec((1, D, 2 * F_MOE), lambda i, be, nb: (be[i], 0, 0)),
            pl.BlockSpec((1, F_MOE, D), lambda i, be, nb: (be[i], 0, 0)),
        ],
        out_specs=pl.BlockSpec((BLK, D2), lambda i, be, nb: (i, 0)),
    )
    return pl.pallas_call(
        _gemm_body,
        grid_spec=grid_spec,
        out_shape=jax.ShapeDtypeStruct((P, D2), jnp.uint32),
        compiler_params=pltpu.CompilerParams(
            dimension_semantics=("arbitrary",)),
    )(be, nb, xs, w_gate_up, w_down)


def _sh_a_body(x_ref, wg_ref, wu_ref, act_ref):
    v = x_ref[...]
    xlo = lax.bitcast_convert_type(v << 16,
                                   jnp.float32).astype(jnp.bfloat16)
    xhi = lax.bitcast_convert_type(v & jnp.uint32(_HI),
                                   jnp.float32).astype(jnp.bfloat16)
    wg = wg_ref[...]
    wu = wu_ref[...]
    g = jnp.dot(xlo, wg[:D2].astype(jnp.bfloat16),
                preferred_element_type=jnp.float32)
    g += jnp.dot(xhi, wg[D2:].astype(jnp.bfloat16),
                 preferred_element_type=jnp.float32)
    u = jnp.dot(xlo, wu[:D2].astype(jnp.bfloat16),
                preferred_element_type=jnp.float32)
    u += jnp.dot(xhi, wu[D2:].astype(jnp.bfloat16),
                 preferred_element_type=jnp.float32)
    act_ref[...] = (_silu(g) * u).astype(jnp.bfloat16)


def _shared_a(xb, sh_gate_up, h):
    bm, bn = 256, 1024
    nh = F_SHARED // bn
    return pl.pallas_call(
        _sh_a_body,
        grid=(T // bm,),
        in_specs=[
            pl.BlockSpec((bm, D2), lambda i: (i, 0)),
            pl.BlockSpec((D, bn), lambda i: (0, h)),
            pl.BlockSpec((D, bn), lambda i: (0, nh + h)),
        ],
        out_specs=pl.BlockSpec((bm, bn), lambda i: (i, 0)),
        out_shape=jax.ShapeDtypeStruct((T, bn), jnp.bfloat16),
        compiler_params=pltpu.CompilerParams(
            dimension_semantics=("arbitrary",)),
    )(xb, sh_gate_up, sh_gate_up)


def _combine_body(a0_ref, a1_ref, wd0_ref, wd1_ref, g0_ref, g1_ref, w_ref,
                  out_ref):
    shared = jnp.dot(a0_ref[...], wd0_ref[...].astype(jnp.bfloat16),
                     preferred_element_type=jnp.float32)
    shared += jnp.dot(a1_ref[...], wd1_ref[...].astype(jnp.bfloat16),
                      preferred_element_type=jnp.float32)
    w = w_ref[...]
    w0 = w[:, 0:1]
    w1 = w[:, 1:2]
    v0 = g0_ref[...]
    v1 = g1_ref[...]
    g0lo = lax.bitcast_convert_type(v0 << 16, jnp.float32)
    g0hi = lax.bitcast_convert_type(v0 & jnp.uint32(_HI), jnp.float32)
    g1lo = lax.bitcast_convert_type(v1 << 16, jnp.float32)
    g1hi = lax.bitcast_convert_type(v1 & jnp.uint32(_HI), jnp.float32)
    out_ref[:, :D2] = shared[:, :D2] + w0 * g0lo + w1 * g1lo
    out_ref[:, D2:] = shared[:, D2:] + w0 * g0hi + w1 * g1hi


def _combine(a0, a1, sh_down, g0, g1, w):
    bm = 256
    fh = F_SHARED // 2
    return pl.pallas_call(
        _combine_body,
        grid=(T // bm,),
        in_specs=[
            pl.BlockSpec((bm, fh), lambda i: (i, 0)),
            pl.BlockSpec((bm, fh), lambda i: (i, 0)),
            pl.BlockSpec((fh, D), lambda i: (0, 0)),
            pl.BlockSpec((fh, D), lambda i: (1, 0)),
            pl.BlockSpec((bm, D2), lambda i: (i, 0)),
            pl.BlockSpec((bm, D2), lambda i: (i, 0)),
            pl.BlockSpec((bm, 2), lambda i: (i, 0)),
        ],
        out_specs=pl.BlockSpec((bm, D), lambda i: (i, 0)),
        out_shape=jax.ShapeDtypeStruct((T, D), jnp.float32),
        compiler_params=pltpu.CompilerParams(
            dimension_semantics=("arbitrary",)),
    )(a0, a1, sh_down, sh_down, g0, g1, w)


def kernel(hidden_states, gate_w, e_bias, w_gate_up, w_down, sh_gate_up,
           sh_down):
    x = hidden_states.reshape(T, D)
    dest, w, be, nb, xb = _router(x, gate_w, e_bias)
    dest_cm = dest.T
    a0 = _shared_a(xb, sh_gate_up, 0)
    xs = _dispatch(xb, dest_cm)
    outs = _grouped_gemm(be.reshape(PBLK), nb.reshape(1), xs,
                         w_gate_up, w_down)
    a1 = _shared_a(xb, sh_gate_up, 1)
    g0, g1 = _gather(outs, dest_cm)
    return _combine(a0, a1, sh_down, g0, g1, w)


# speedup vs baseline: 1.0402x; 1.0402x over previous
import functools

import jax
import jax.numpy as jnp
from jax import lax
from jax.experimental import pallas as pl
from jax.experimental.pallas import tpu as pltpu
from jax.experimental.pallas import tpu_sc as plsc

T = 2048
D = 2048
F_MOE = 1024
E = 8
N_SHARED = 2
F_SHARED = F_MOE * N_SHARED
SCALE = 2.5

BLK = 256
PBLK = (2 * T) // BLK + E
P = PBLK * BLK

NC = 2
NS = 16
NW = NC * NS
TPW = T // NW
CH = 32
D2 = D // 2
_HI = 0xFFFF0000
_RB = 0x8000


def _silu(x):
    return x * jax.nn.sigmoid(x)


def _router_body(x_ref, gw_ref, eb_ref, dest_ref, w_ref, blk_ref, nblk_ref,
                 xb_ref, excl_ref, m_ref):
    x = x_ref[...]
    xb = x.astype(jnp.bfloat16)
    xlo = lax.bitcast_convert_type(x[:, :D2], jnp.uint32)
    xhi = lax.bitcast_convert_type(x[:, D2:], jnp.uint32)
    xb_ref[...] = (((xlo + jnp.uint32(_RB)) >> 16)
                   | ((xhi + jnp.uint32(_RB)) & jnp.uint32(_HI)))
    gwb = gw_ref[...].astype(jnp.bfloat16)
    logits = lax.dot_general(
        xb, gwb, (((1,), (1,)), ((), ())),
        preferred_element_type=jnp.float32)
    scores = jax.nn.sigmoid(logits)
    corrected = scores + eb_ref[...]

    iota_e = lax.broadcasted_iota(jnp.int32, (T, E), 1)
    m1 = jnp.max(corrected, axis=1, keepdims=True)
    msk1 = corrected == m1
    sel1 = jnp.min(jnp.where(msk1, iota_e, jnp.int32(E)), axis=1,
                   keepdims=True)
    oh1 = (iota_e == sel1).astype(jnp.float32)
    corrected2 = jnp.where(oh1 > 0, -jnp.inf, corrected)
    m2 = jnp.max(corrected2, axis=1, keepdims=True)
    msk2 = corrected2 == m2
    sel2 = jnp.min(jnp.where(msk2, iota_e, jnp.int32(E)), axis=1,
                   keepdims=True)
    oh2 = (iota_e == sel2).astype(jnp.float32)

    w1 = jnp.sum(oh1 * scores, axis=1, keepdims=True)
    w2 = jnp.sum(oh2 * scores, axis=1, keepdims=True)
    denom = w1 + w2 + jnp.float32(1e-20)
    w1 = w1 / denom * jnp.float32(SCALE)
    w2 = w2 / denom * jnp.float32(SCALE)
    w_ref[...] = jnp.concatenate([w1, w2], axis=1)

    m_ref[...] = oh1 + oh2
    chr_ = 512
    tri = (lax.broadcasted_iota(jnp.int32, (chr_, chr_), 0)
           > lax.broadcasted_iota(jnp.int32, (chr_, chr_), 1)
           ).astype(jnp.float32)

    def body(c, prefix):
        blk = m_ref[pl.ds(c * chr_, chr_), :]
        part = lax.dot_general(
            tri, blk, (((1,), (0,)), ((), ())),
            preferred_element_type=jnp.float32,
            precision=lax.Precision.HIGHEST)
        excl_ref[pl.ds(c * chr_, chr_), :] = part + prefix
        return prefix + jnp.sum(blk, axis=0, keepdims=True)

    counts = lax.fori_loop(0, T // chr_, body, jnp.zeros((1, E), jnp.float32))
    excl = excl_ref[...]

    pc = jnp.ceil(counts / BLK) * BLK
    upper = (lax.broadcasted_iota(jnp.int32, (E, E), 0)
             < lax.broadcasted_iota(jnp.int32, (E, E), 1)).astype(jnp.float32)
    off = lax.dot_general(pc, upper, (((1,), (0,)), ((), ())),
                          preferred_element_type=jnp.float32)

    d1 = jnp.sum(oh1 * (off + excl), axis=1, keepdims=True)
    d2 = jnp.sum(oh2 * (off + excl), axis=1, keepdims=True)
    dest_ref[...] = jnp.concatenate([d1, d2], axis=1).astype(jnp.int32)

    starts = (lax.broadcasted_iota(jnp.int32, (1, PBLK), 1)
              * BLK).astype(jnp.float32)
    offc = jnp.reshape(off, (E, 1))
    be = jnp.sum((starts >= offc).astype(jnp.int32), axis=0,
                 keepdims=True) - 1
    blk_ref[...] = be
    used = jnp.sum(pc) / BLK
    nblk_ref[...] = used.astype(jnp.int32).reshape(1, 1)


def _router(x, gate_w, e_bias):
    return pl.pallas_call(
        _router_body,
        out_shape=(
            jax.ShapeDtypeStruct((T, 2), jnp.int32),
            jax.ShapeDtypeStruct((T, 2), jnp.float32),
            jax.ShapeDtypeStruct((1, PBLK), jnp.int32),
            jax.ShapeDtypeStruct((1, 1), jnp.int32),
            jax.ShapeDtypeStruct((T, D2), jnp.uint32),
        ),
        scratch_shapes=[pltpu.VMEM((T, E), jnp.float32),
                        pltpu.VMEM((T, E), jnp.float32)],
    )(x, gate_w, e_bias.reshape(1, E))


def _sc_mesh():
    return plsc.VectorSubcoreMesh(core_axis_name="c", subcore_axis_name="s",
                                  num_cores=NC, num_subcores=NS)


def _dispatch(x, dest_cm):
    @functools.partial(
        pl.kernel,
        mesh=_sc_mesh(),
        out_type=jax.ShapeDtypeStruct((P, D2), jnp.uint32),
        scratch_types=[
            pltpu.VMEM((CH,), jnp.int32),
            pltpu.VMEM((CH,), jnp.int32),
            pltpu.VMEM((CH, D2), jnp.uint32),
            pltpu.SemaphoreType.DMA,
        ],
    )
    def k(x_hbm, dest_hbm, xs_hbm, idx0_v, idx1_v, rows_v, sem):
        wid = lax.axis_index("s") * NC + lax.axis_index("c")
        base = wid * TPW
        for c in range(TPW // CH):
            off = base + c * CH
            pltpu.sync_copy(dest_hbm.at[0, pl.ds(off, CH)], idx0_v)
            pltpu.sync_copy(dest_hbm.at[1, pl.ds(off, CH)], idx1_v)
            pltpu.sync_copy(x_hbm.at[pl.ds(off, CH)], rows_v)
            a = pltpu.async_copy(rows_v, xs_hbm.at[idx0_v], sem)
            b = pltpu.async_copy(rows_v, xs_hbm.at[idx1_v], sem)
            a.wait()
            b.wait()

    return k(x, dest_cm)


def _gather(outs, dest_cm):
    @functools.partial(
        pl.kernel,
        mesh=_sc_mesh(),
        out_type=(
            jax.ShapeDtypeStruct((T, D2), jnp.uint32),
            jax.ShapeDtypeStruct((T, D2), jnp.uint32),
        ),
        scratch_types=[
            pltpu.VMEM((CH,), jnp.int32),
            pltpu.VMEM((CH, D2), jnp.uint32),
            pltpu.SemaphoreType.DMA,
        ],
    )
    def k(outs_hbm, dest_hbm, g0_hbm, g1_hbm, idx_v, rows_v, sem):
        wid = lax.axis_index("s") * NC + lax.axis_index("c")
        base = wid * TPW
        for c in range(TPW // CH):
            off = base + c * CH
            pltpu.sync_copy(dest_hbm.at[0, pl.ds(off, CH)], idx_v)
            pltpu.async_copy(outs_hbm.at[idx_v], rows_v, sem).wait()
            pltpu.sync_copy(rows_v, g0_hbm.at[pl.ds(off, CH)])
            pltpu.sync_copy(dest_hbm.at[1, pl.ds(off, CH)], idx_v)
            pltpu.async_copy(outs_hbm.at[idx_v], rows_v, sem).wait()
            pltpu.sync_copy(rows_v, g1_hbm.at[pl.ds(off, CH)])

    return k(outs, dest_cm)


def _gemm_body(be_ref, nb_ref, x_ref, wgu_ref, wdn_ref, out_ref):
    i = pl.program_id(0)

    @pl.when(i < nb_ref[0])
    def _():
        v = x_ref[...]
        xlo = lax.bitcast_convert_type(v << 16, jnp.float32)
        xhi = lax.bitcast_convert_type(v & jnp.uint32(_HI), jnp.float32)
        wgu = wgu_ref[0]
        gu = jnp.dot(xlo, wgu[:D2], preferred_element_type=jnp.float32)
        gu += jnp.dot(xhi, wgu[D2:], preferred_element_type=jnp.float32)
        g = gu[:, :F_MOE]
        u = gu[:, F_MOE:]
        act = _silu(g) * u
        o = jnp.dot(act, wdn_ref[0],
                    preferred_element_type=jnp.float32)
        olo = lax.bitcast_convert_type(o[:, :D2], jnp.uint32)
        ohi = lax.bitcast_convert_type(o[:, D2:], jnp.uint32)
        out_ref[...] = (((olo + jnp.uint32(_RB)) >> 16)
                        | ((ohi + jnp.uint32(_RB)) & jnp.uint32(_HI)))


def _grouped_gemm(be, nb, xs, w_gate_up, w_down):
    grid_spec = pltpu.PrefetchScalarGridSpec(
        num_scalar_prefetch=2,
        grid=(PBLK,),
        in_specs=[
            pl.BlockSpec((BLK, D2),
                         lambda i, be, nb: (jnp.minimum(i, nb[0] - 1), 0)),
            pl.BlockSpec((1, D, 2 * F_MOE), lambda i, be, nb: (be[i], 0, 0)),
            pl.BlockSpec((1, F_MOE, D), lambda i, be, nb: (be[i], 0, 0)),
        ],
        out_specs=pl.BlockSpec((BLK, D2), lambda i, be, nb: (i, 0)),
    )
    return pl.pallas_call(
        _gemm_body,
        grid_spec=grid_spec,
        out_shape=jax.ShapeDtypeStruct((P, D2), jnp.uint32),
        compiler_params=pltpu.CompilerParams(
            dimension_semantics=("arbitrary",)),
    )(be, nb, xs, w_gate_up, w_down)


def _sh_a_body(x_ref, wg_ref, wu_ref, act_ref):
    v = x_ref[...]
    xlo = lax.bitcast_convert_type(v << 16, jnp.float32)
    xhi = lax.bitcast_convert_type(v & jnp.uint32(_HI), jnp.float32)
    wg = wg_ref[...]
    wu = wu_ref[...]
    g = jnp.dot(xlo, wg[:D2], preferred_element_type=jnp.float32)
    g += jnp.dot(xhi, wg[D2:], preferred_element_type=jnp.float32)
    u = jnp.dot(xlo, wu[:D2], preferred_element_type=jnp.float32)
    u += jnp.dot(xhi, wu[D2:], preferred_element_type=jnp.float32)
    act_ref[...] = (_silu(g) * u).astype(jnp.bfloat16)


def _shared_a(xb, sh_gate_up, h):
    bm, bn = 256, 1024
    nh = F_SHARED // bn
    return pl.pallas_call(
        _sh_a_body,
        grid=(T // bm,),
        in_specs=[
            pl.BlockSpec((bm, D2), lambda i: (i, 0)),
            pl.BlockSpec((D, bn), lambda i: (0, h)),
            pl.BlockSpec((D, bn), lambda i: (0, nh + h)),
        ],
        out_specs=pl.BlockSpec((bm, bn), lambda i: (i, 0)),
        out_shape=jax.ShapeDtypeStruct((T, bn), jnp.bfloat16),
        compiler_params=pltpu.CompilerParams(
            dimension_semantics=("arbitrary",)),
    )(xb, sh_gate_up, sh_gate_up)


def _combine_body(a0_ref, a1_ref, wd0_ref, wd1_ref, g0_ref, g1_ref, w_ref,
                  out_ref):
    shared = jnp.dot(a0_ref[...].astype(jnp.float32), wd0_ref[...],
                     preferred_element_type=jnp.float32)
    shared += jnp.dot(a1_ref[...].astype(jnp.float32), wd1_ref[...],
                      preferred_element_type=jnp.float32)
    w = w_ref[...]
    w0 = w[:, 0:1]
    w1 = w[:, 1:2]
    v0 = g0_ref[...]
    v1 = g1_ref[...]
    g0lo = lax.bitcast_convert_type(v0 << 16, jnp.float32)
    g0hi = lax.bitcast_convert_type(v0 & jnp.uint32(_HI), jnp.float32)
    g1lo = lax.bitcast_convert_type(v1 << 16, jnp.float32)
    g1hi = lax.bitcast_convert_type(v1 & jnp.uint32(_HI), jnp.float32)
    out_ref[:, :D2] = shared[:, :D2] + w0 * g0lo + w1 * g1lo
    out_ref[:, D2:] = shared[:, D2:] + w0 * g0hi + w1 * g1hi


def _combine(a0, a1, sh_down, g0, g1, w):
    bm = 256
    fh = F_SHARED // 2
    return pl.pallas_call(
        _combine_body,
        grid=(T // bm,),
        in_specs=[
            pl.BlockSpec((bm, fh), lambda i: (i, 0)),
            pl.BlockSpec((bm, fh), lambda i: (i, 0)),
            pl.BlockSpec((fh, D), lambda i: (0, 0)),
            pl.BlockSpec((fh, D), lambda i: (1, 0)),
            pl.BlockSpec((bm, D2), lambda i: (i, 0)),
            pl.BlockSpec((bm, D2), lambda i: (i, 0)),
            pl.BlockSpec((bm, 2), lambda i: (i, 0)),
        ],
        out_specs=pl.BlockSpec((bm, D), lambda i: (i, 0)),
        out_shape=jax.ShapeDtypeStruct((T, D), jnp.float32),
        compiler_params=pltpu.CompilerParams(
            dimension_semantics=("arbitrary",)),
    )(a0, a1, sh_down, sh_down, g0, g1, w)


def kernel(hidden_states, gate_w, e_bias, w_gate_up, w_down, sh_gate_up,
           sh_down):
    x = hidden_states.reshape(T, D)
    dest, w, be, nb, xb = _router(x, gate_w, e_bias)
    dest_cm = dest.T
    a0 = _shared_a(xb, sh_gate_up, 0)
    xs = _dispatch(xb, dest_cm)
    outs = _grouped_gemm(be.reshape(PBLK), nb.reshape(1), xs,
                         w_gate_up, w_down)
    a1 = _shared_a(xb, sh_gate_up, 1)
    g0, g1 = _gather(outs, dest_cm)
    return _combine(a0, a1, sh_down, g0, g1, w)


# speedup vs baseline: 1.0438x; 1.0035x over previous
import functools

import jax
import jax.numpy as jnp
from jax import lax
from jax.experimental import pallas as pl
from jax.experimental.pallas import tpu as pltpu
from jax.experimental.pallas import tpu_sc as plsc

T = 2048
D = 2048
F_MOE = 1024
E = 8
N_SHARED = 2
F_SHARED = F_MOE * N_SHARED
SCALE = 2.5

BLK = 256
PBLK = (2 * T) // BLK + E
P = PBLK * BLK

NC = 2
NS = 16
NW = NC * NS
TPW = T // NW
CH = 64
D2 = D // 2
_HI = 0xFFFF0000
_RB = 0x8000


def _silu(x):
    return x * jax.nn.sigmoid(x)


def _router_body(x_ref, gw_ref, eb_ref, dest_ref, w_ref, blk_ref, nblk_ref,
                 xb_ref, excl_ref, m_ref):
    x = x_ref[...]
    xb = x.astype(jnp.bfloat16)
    xlo = lax.bitcast_convert_type(x[:, :D2], jnp.uint32)
    xhi = lax.bitcast_convert_type(x[:, D2:], jnp.uint32)
    xb_ref[...] = (((xlo + jnp.uint32(_RB)) >> 16)
                   | ((xhi + jnp.uint32(_RB)) & jnp.uint32(_HI)))
    gwb = gw_ref[...].astype(jnp.bfloat16)
    logits = lax.dot_general(
        xb, gwb, (((1,), (1,)), ((), ())),
        preferred_element_type=jnp.float32)
    scores = jax.nn.sigmoid(logits)
    corrected = scores + eb_ref[...]

    iota_e = lax.broadcasted_iota(jnp.int32, (T, E), 1)
    m1 = jnp.max(corrected, axis=1, keepdims=True)
    msk1 = corrected == m1
    sel1 = jnp.min(jnp.where(msk1, iota_e, jnp.int32(E)), axis=1,
                   keepdims=True)
    oh1 = (iota_e == sel1).astype(jnp.float32)
    corrected2 = jnp.where(oh1 > 0, -jnp.inf, corrected)
    m2 = jnp.max(corrected2, axis=1, keepdims=True)
    msk2 = corrected2 == m2
    sel2 = jnp.min(jnp.where(msk2, iota_e, jnp.int32(E)), axis=1,
                   keepdims=True)
    oh2 = (iota_e == sel2).astype(jnp.float32)

    w1 = jnp.sum(oh1 * scores, axis=1, keepdims=True)
    w2 = jnp.sum(oh2 * scores, axis=1, keepdims=True)
    denom = w1 + w2 + jnp.float32(1e-20)
    w1 = w1 / denom * jnp.float32(SCALE)
    w2 = w2 / denom * jnp.float32(SCALE)
    w_ref[...] = jnp.concatenate([w1, w2], axis=1)

    m_ref[...] = oh1 + oh2
    chr_ = 512
    tri = (lax.broadcasted_iota(jnp.int32, (chr_, chr_), 0)
           > lax.broadcasted_iota(jnp.int32, (chr_, chr_), 1)
           ).astype(jnp.float32)

    def body(c, prefix):
        blk = m_ref[pl.ds(c * chr_, chr_), :]
        part = lax.dot_general(
            tri, blk, (((1,), (0,)), ((), ())),
            preferred_element_type=jnp.float32,
            precision=lax.Precision.HIGHEST)
        excl_ref[pl.ds(c * chr_, chr_), :] = part + prefix
        return prefix + jnp.sum(blk, axis=0, keepdims=True)

    counts = lax.fori_loop(0, T // chr_, body, jnp.zeros((1, E), jnp.float32))
    excl = excl_ref[...]

    pc = jnp.ceil(counts / BLK) * BLK
    upper = (lax.broadcasted_iota(jnp.int32, (E, E), 0)
             < lax.broadcasted_iota(jnp.int32, (E, E), 1)).astype(jnp.float32)
    off = lax.dot_general(pc, upper, (((1,), (0,)), ((), ())),
                          preferred_element_type=jnp.float32)

    d1 = jnp.sum(oh1 * (off + excl), axis=1, keepdims=True)
    d2 = jnp.sum(oh2 * (off + excl), axis=1, keepdims=True)
    dest_ref[...] = jnp.concatenate([d1, d2], axis=1).astype(jnp.int32)

    starts = (lax.broadcasted_iota(jnp.int32, (1, PBLK), 1)
              * BLK).astype(jnp.float32)
    offc = jnp.reshape(off, (E, 1))
    be = jnp.sum((starts >= offc).astype(jnp.int32), axis=0,
                 keepdims=True) - 1
    blk_ref[...] = be
    used = jnp.sum(pc) / BLK
    nblk_ref[...] = used.astype(jnp.int32).reshape(1, 1)


def _router(x, gate_w, e_bias):
    return pl.pallas_call(
        _router_body,
        out_shape=(
            jax.ShapeDtypeStruct((T, 2), jnp.int32),
            jax.ShapeDtypeStruct((T, 2), jnp.float32),
            jax.ShapeDtypeStruct((1, PBLK), jnp.int32),
            jax.ShapeDtypeStruct((1, 1), jnp.int32),
            jax.ShapeDtypeStruct((T, D2), jnp.uint32),
        ),
        scratch_shapes=[pltpu.VMEM((T, E), jnp.float32),
                        pltpu.VMEM((T, E), jnp.float32)],
    )(x, gate_w, e_bias.reshape(1, E))


def _sc_mesh():
    return plsc.VectorSubcoreMesh(core_axis_name="c", subcore_axis_name="s",
                                  num_cores=NC, num_subcores=NS)


def _dispatch(x, dest_cm):
    @functools.partial(
        pl.kernel,
        mesh=_sc_mesh(),
        out_type=jax.ShapeDtypeStruct((P, D2), jnp.uint32),
        scratch_types=[
            pltpu.VMEM((CH,), jnp.int32),
            pltpu.VMEM((CH,), jnp.int32),
            pltpu.VMEM((CH, D2), jnp.uint32),
            pltpu.SemaphoreType.DMA,
        ],
    )
    def k(x_hbm, dest_hbm, xs_hbm, idx0_v, idx1_v, rows_v, sem):
        wid = lax.axis_index("s") * NC + lax.axis_index("c")
        base = wid * TPW
        for c in range(TPW // CH):
            off = base + c * CH
            pltpu.sync_copy(dest_hbm.at[0, pl.ds(off, CH)], idx0_v)
            pltpu.sync_copy(dest_hbm.at[1, pl.ds(off, CH)], idx1_v)
            pltpu.sync_copy(x_hbm.at[pl.ds(off, CH)], rows_v)
            a = pltpu.async_copy(rows_v, xs_hbm.at[idx0_v], sem)
            b = pltpu.async_copy(rows_v, xs_hbm.at[idx1_v], sem)
            a.wait()
            b.wait()

    return k(x, dest_cm)


def _gather(outs, dest_cm):
    @functools.partial(
        pl.kernel,
        mesh=_sc_mesh(),
        out_type=(
            jax.ShapeDtypeStruct((T, D2), jnp.uint32),
            jax.ShapeDtypeStruct((T, D2), jnp.uint32),
        ),
        scratch_types=[
            pltpu.VMEM((CH,), jnp.int32),
            pltpu.VMEM((CH, D2), jnp.uint32),
            pltpu.SemaphoreType.DMA,
        ],
    )
    def k(outs_hbm, dest_hbm, g0_hbm, g1_hbm, idx_v, rows_v, sem):
        wid = lax.axis_index("s") * NC + lax.axis_index("c")
        base = wid * TPW
        for c in range(TPW // CH):
            off = base + c * CH
            pltpu.sync_copy(dest_hbm.at[0, pl.ds(off, CH)], idx_v)
            pltpu.async_copy(outs_hbm.at[idx_v], rows_v, sem).wait()
            pltpu.sync_copy(rows_v, g0_hbm.at[pl.ds(off, CH)])
            pltpu.sync_copy(dest_hbm.at[1, pl.ds(off, CH)], idx_v)
            pltpu.async_copy(outs_hbm.at[idx_v], rows_v, sem).wait()
            pltpu.sync_copy(rows_v, g1_hbm.at[pl.ds(off, CH)])

    return k(outs, dest_cm)


def _gemm_body(be_ref, nb_ref, x_ref, wgu_ref, wdn_ref, out_ref):
    i = pl.program_id(0)

    @pl.when(i < nb_ref[0])
    def _():
        v = x_ref[...]
        xlo = lax.bitcast_convert_type(v << 16, jnp.float32)
        xhi = lax.bitcast_convert_type(v & jnp.uint32(_HI), jnp.float32)
        wgu = wgu_ref[0]
        gu = jnp.dot(xlo, wgu[:D2], preferred_element_type=jnp.float32)
        gu += jnp.dot(xhi, wgu[D2:], preferred_element_type=jnp.float32)
        g = gu[:, :F_MOE]
        u = gu[:, F_MOE:]
        act = _silu(g) * u
        o = jnp.dot(act, wdn_ref[0],
                    preferred_element_type=jnp.float32)
        olo = lax.bitcast_convert_type(o[:, :D2], jnp.uint32)
        ohi = lax.bitcast_convert_type(o[:, D2:], jnp.uint32)
        out_ref[...] = (((olo + jnp.uint32(_RB)) >> 16)
                        | ((ohi + jnp.uint32(_RB)) & jnp.uint32(_HI)))


def _grouped_gemm(be, nb, xs, w_gate_up, w_down):
    grid_spec = pltpu.PrefetchScalarGridSpec(
        num_scalar_prefetch=2,
        grid=(PBLK,),
        in_specs=[
            pl.BlockSpec((BLK, D2),
                         lambda i, be, nb: (jnp.minimum(i, nb[0] - 1), 0)),
            pl.BlockSpec((1, D, 2 * F_MOE), lambda i, be, nb: (be[i], 0, 0)),
            pl.BlockSpec((1, F_MOE, D), lambda i, be, nb: (be[i], 0, 0)),
        ],
        out_specs=pl.BlockSpec((BLK, D2), lambda i, be, nb: (i, 0)),
    )
    return pl.pallas_call(
        _gemm_body,
        grid_spec=grid_spec,
        out_shape=jax.ShapeDtypeStruct((P, D2), jnp.uint32),
        compiler_params=pltpu.CompilerParams(
            dimension_semantics=("arbitrary",)),
    )(be, nb, xs, w_gate_up, w_down)


def _sh_a_body(x_ref, wg_ref, wu_ref, act_ref):
    v = x_ref[...]
    xlo = lax.bitcast_convert_type(v << 16, jnp.float32)
    xhi = lax.bitcast_convert_type(v & jnp.uint32(_HI), jnp.float32)
    wg = wg_ref[...]
    wu = wu_ref[...]
    g = jnp.dot(xlo, wg[:D2], preferred_element_type=jnp.float32)
    g += jnp.dot(xhi, wg[D2:], preferred_element_type=jnp.float32)
    u = jnp.dot(xlo, wu[:D2], preferred_element_type=jnp.float32)
    u += jnp.dot(xhi, wu[D2:], preferred_element_type=jnp.float32)
    act_ref[...] = (_silu(g) * u).astype(jnp.bfloat16)


def _shared_a(xb, sh_gate_up, h):
    bm, bn = 256, 1024
    nh = F_SHARED // bn
    return pl.pallas_call(
        _sh_a_body,
        grid=(T // bm,),
        in_specs=[
            pl.BlockSpec((bm, D2), lambda i: (i, 0)),
            pl.BlockSpec((D, bn), lambda i: (0, h)),
            pl.BlockSpec((D, bn), lambda i: (0, nh + h)),
        ],
        out_specs=pl.BlockSpec((bm, bn), lambda i: (i, 0)),
        out_shape=jax.ShapeDtypeStruct((T, bn), jnp.bfloat16),
        compiler_params=pltpu.CompilerParams(
            dimension_semantics=("arbitrary",)),
    )(xb, sh_gate_up, sh_gate_up)


def _combine_body(a0_ref, a1_ref, wd0_ref, wd1_ref, g0_ref, g1_ref, w_ref,
                  out_ref):
    shared = jnp.dot(a0_ref[...].astype(jnp.float32), wd0_ref[...],
                     preferred_element_type=jnp.float32)
    shared += jnp.dot(a1_ref[...].astype(jnp.float32), wd1_ref[...],
                      preferred_element_type=jnp.float32)
    w = w_ref[...]
    w0 = w[:, 0:1]
    w1 = w[:, 1:2]
    v0 = g0_ref[...]
    v1 = g1_ref[...]
    g0lo = lax.bitcast_convert_type(v0 << 16, jnp.float32)
    g0hi = lax.bitcast_convert_type(v0 & jnp.uint32(_HI), jnp.float32)
    g1lo = lax.bitcast_convert_type(v1 << 16, jnp.float32)
    g1hi = lax.bitcast_convert_type(v1 & jnp.uint32(_HI), jnp.float32)
    out_ref[:, :D2] = shared[:, :D2] + w0 * g0lo + w1 * g1lo
    out_ref[:, D2:] = shared[:, D2:] + w0 * g0hi + w1 * g1hi


def _combine(a0, a1, sh_down, g0, g1, w):
    bm = 256
    fh = F_SHARED // 2
    return pl.pallas_call(
        _combine_body,
        grid=(T // bm,),
        in_specs=[
            pl.BlockSpec((bm, fh), lambda i: (i, 0)),
            pl.BlockSpec((bm, fh), lambda i: (i, 0)),
            pl.BlockSpec((fh, D), lambda i: (0, 0)),
            pl.BlockSpec((fh, D), lambda i: (1, 0)),
            pl.BlockSpec((bm, D2), lambda i: (i, 0)),
            pl.BlockSpec((bm, D2), lambda i: (i, 0)),
            pl.BlockSpec((bm, 2), lambda i: (i, 0)),
        ],
        out_specs=pl.BlockSpec((bm, D), lambda i: (i, 0)),
        out_shape=jax.ShapeDtypeStruct((T, D), jnp.float32),
        compiler_params=pltpu.CompilerParams(
            dimension_semantics=("arbitrary",)),
    )(a0, a1, sh_down, sh_down, g0, g1, w)


def kernel(hidden_states, gate_w, e_bias, w_gate_up, w_down, sh_gate_up,
           sh_down):
    x = hidden_states.reshape(T, D)
    dest, w, be, nb, xb = _router(x, gate_w, e_bias)
    dest_cm = dest.T
    a0 = _shared_a(xb, sh_gate_up, 0)
    xs = _dispatch(xb, dest_cm)
    outs = _grouped_gemm(be.reshape(PBLK), nb.reshape(1), xs,
                         w_gate_up, w_down)
    a1 = _shared_a(xb, sh_gate_up, 1)
    g0, g1 = _gather(outs, dest_cm)
    return _combine(a0, a1, sh_down, g0, g1, w)
